# baseline (speedup 1.0000x reference)
import functools

import jax
import jax.numpy as jnp
from jax import lax
from jax.experimental import pallas as pl
from jax.experimental.pallas import tpu as pltpu
from jax.experimental.pallas import tpu_sc as plsc

_NBUF = 16
_ROWS = 128


def _ring_body(x_hbm, o_hbm):
    r, d = x_hbm.shape
    g = r // _ROWS
    half = _NBUF // 2

    def scoped(bufs, in_sems, out_sems):
        def in_copy(i, slot):
            return pltpu.make_async_copy(
                x_hbm.at[pl.ds(i * _ROWS, _ROWS)], bufs.at[slot], in_sems.at[slot]
            )

        def out_copy(i, slot):
            return pltpu.make_async_copy(
                bufs.at[slot], o_hbm.at[pl.ds(i * _ROWS, _ROWS)], out_sems.at[slot]
            )

        for j in range(half):
            in_copy(j, j).start()

        def step(i, carry):
            slot = lax.rem(i, _NBUF)
            in_copy(i, slot).wait()
            out_copy(i, slot).start()
            j = i + half
            slot_j = lax.rem(j, _NBUF)

            @pl.when(j < g)
            def _():
                @pl.when(j >= _NBUF)
                def _():
                    out_copy(j - _NBUF, slot_j).wait()

                in_copy(j, slot_j).start()

            return carry

        lax.fori_loop(0, g, step, 0)
        for k in range(_NBUF):
            i = g - _NBUF + k
            out_copy(i, i % _NBUF).wait()

    pl.run_scoped(
        scoped,
        pltpu.VMEM((_NBUF, _ROWS, x_hbm.shape[1]), jnp.float32),
        pltpu.SemaphoreType.DMA((_NBUF,)),
        pltpu.SemaphoreType.DMA((_NBUF,)),
    )


def _tc_copy(x2d):
    return pl.pallas_call(
        _ring_body,
        in_specs=[pl.BlockSpec(memory_space=pl.ANY)],
        out_specs=pl.BlockSpec(memory_space=pl.ANY),
        out_shape=jax.ShapeDtypeStruct(x2d.shape, x2d.dtype),
    )(x2d)


def _make_sc_scatter(b, s, d, n, chunk=16):
    nc, ns = 2, 16
    mesh = plsc.VectorSubcoreMesh(
        core_axis_name="c", subcore_axis_name="s", num_cores=nc, num_subcores=ns
    )
    nworkers = (b * n) // chunk

    @functools.partial(
        pl.kernel,
        out_type=(),
        mesh=mesh,
        scratch_types=[
            pltpu.VMEM((chunk,), jnp.int32),
            pltpu.VMEM((chunk, d), jnp.float32),
            pltpu.SemaphoreType.DMA,
        ],
    )
    def sc_scatter(out_ref, vals_hbm, idx_hbm, idx_v, rows_v, sem):
        wid = lax.axis_index("s") * nc + lax.axis_index("c")

        @pl.when(wid < nworkers)
        def _():
            t0 = wid * chunk
            batch = t0 // n
            i0 = t0 % n
            pltpu.sync_copy(idx_hbm.at[pl.ds(i0, chunk)], idx_v)
            pltpu.sync_copy(vals_hbm.at[pl.ds(i0, chunk)], rows_v)
            flat = idx_v[...] + batch * s
            pltpu.async_copy(rows_v, out_ref.at[flat], sem).wait()

    return sc_scatter


def kernel(x, replace_vals, replace_idx):
    b, s, d = x.shape
    n = replace_vals.shape[0]
    x2d = x.reshape(b * s, d)
    y = _tc_copy(x2d)
    y_ref = jax.new_ref(y)
    _make_sc_scatter(b, s, d, n)(y_ref, replace_vals, replace_idx)
    return jax.freeze(y_ref).reshape(b, s, d)


# speedup vs baseline: 1.0052x; 1.0052x over previous
import functools

import jax
import jax.numpy as jnp
from jax import lax
from jax.experimental import pallas as pl
from jax.experimental.pallas import tpu as pltpu
from jax.experimental.pallas import tpu_sc as plsc


def _copy_body(x_ref, o_ref):
    o_ref[...] = x_ref[...]


def _tc_copy(x2d, blk):
    r, d = x2d.shape
    return pl.pallas_call(
        _copy_body,
        grid=(r // blk,),
        in_specs=[pl.BlockSpec((blk, d), lambda i: (i, 0))],
        out_specs=pl.BlockSpec((blk, d), lambda i: (i, 0)),
        out_shape=jax.ShapeDtypeStruct(x2d.shape, x2d.dtype),
    )(x2d)


def _make_sc_scatter(b, s, d, n, chunk=16):
    nc, ns = 2, 16
    mesh = plsc.VectorSubcoreMesh(
        core_axis_name="c", subcore_axis_name="s", num_cores=nc, num_subcores=ns
    )
    ntasks = b * n
    nworkers = ntasks // chunk

    @functools.partial(
        pl.kernel,
        out_type=(),
        mesh=mesh,
        scratch_types=[
            pltpu.VMEM((chunk,), jnp.int32),
            pltpu.VMEM((chunk, d), jnp.float32),
            pltpu.SemaphoreType.DMA,
        ],
    )
    def sc_scatter(out_ref, vals_hbm, idx_hbm, idx_v, rows_v, sem):
        wid = lax.axis_index("s") * nc + lax.axis_index("c")

        @pl.when(wid < nworkers)
        def _():
            t0 = wid * chunk
            batch = t0 // n
            i0 = t0 % n
            vals_cp = pltpu.make_async_copy(
                vals_hbm.at[pl.ds(i0, chunk)], rows_v, sem
            )
            vals_cp.start()
            pltpu.sync_copy(idx_hbm.at[pl.ds(i0, chunk)], idx_v)
            flat = idx_v[...] + batch * s
            vals_cp.wait()
            pltpu.async_copy(rows_v, out_ref.at[flat], sem).wait()

    return sc_scatter


def kernel(x, replace_vals, replace_idx):
    b, s, d = x.shape
    n = replace_vals.shape[0]
    x2d = x.reshape(b * s, d)
    y = _tc_copy(x2d, blk=512)
    y_ref = jax.new_ref(y)
    _make_sc_scatter(b, s, d, n)(y_ref, replace_vals, replace_idx)
    return jax.freeze(y_ref).reshape(b, s, d)
